# baseline (speedup 1.0000x reference)
import jax
import jax.numpy as jnp
from jax import lax
from jax.experimental import pallas as pl
from jax.experimental.pallas import tpu as pltpu
from jax.experimental.pallas import tpu_sc as plsc

N, DEG, D, COORD = 10000, 32, 128, 3
BN = 400


def _sc_body(src_hbm, out_hbm, buf):
    wid = lax.axis_index("s") * 2 + lax.axis_index("c")

    @pl.when(wid == 0)
    def _():
        pltpu.sync_copy(src_hbm.at[pl.ds(0, 16)], buf)
        pltpu.sync_copy(buf, out_hbm)


def _tc_body(hh_ref, e_ref, W1_ref, b1_ref, W2_ref, b2_ref, h_ref):
    ef = jnp.sum(e_ref[...], axis=1)
    hh = hh_ref[...]
    W1 = W1_ref[...]
    h1 = (jnp.dot(hh, W1[:D, :], preferred_element_type=jnp.float32)
          + jnp.dot(ef, W1[D:, :], preferred_element_type=jnp.float32)
          + b1_ref[...])
    h1 = h1 * jax.nn.sigmoid(h1)
    h_ref[...] = (hh
                  + jnp.dot(h1, W2_ref[...], preferred_element_type=jnp.float32)
                  + b2_ref[...])


def kernel(x, hh, trans, edge_feature, W1, b1, W2, b2):
    mesh = plsc.VectorSubcoreMesh(core_axis_name="c", subcore_axis_name="s")
    probe = pl.kernel(
        _sc_body,
        out_type=jax.ShapeDtypeStruct((16,), jnp.float32),
        mesh=mesh,
        scratch_types=[pltpu.VMEM((16,), jnp.float32)],
        compiler_params=pltpu.CompilerParams(needs_layout_passes=False),
    )(hh.reshape(-1))

    h = pl.pallas_call(
        _tc_body,
        grid=(N // BN,),
        in_specs=[
            pl.BlockSpec((BN, D), lambda i: (i, 0)),
            pl.BlockSpec((BN, DEG, D), lambda i: (i, 0, 0)),
            pl.BlockSpec((2 * D, D), lambda i: (0, 0)),
            pl.BlockSpec((1, D), lambda i: (0, 0)),
            pl.BlockSpec((D, D), lambda i: (0, 0)),
            pl.BlockSpec((1, D), lambda i: (0, 0)),
        ],
        out_specs=pl.BlockSpec((BN, D), lambda i: (i, 0)),
        out_shape=jax.ShapeDtypeStruct((N, D), jnp.float32),
        compiler_params=pltpu.CompilerParams(
            dimension_semantics=("arbitrary",),
        ),
    )(hh, edge_feature, W1, b1.reshape(1, D), W2, b2.reshape(1, D))
    coord = jnp.zeros((N, COORD), jnp.float32) + probe[0]
    return coord, h


# speedup vs baseline: 1.2740x; 1.2740x over previous
import jax
import jax.numpy as jnp
from jax import lax
from jax.experimental import pallas as pl
from jax.experimental.pallas import tpu as pltpu

N, DEG, D, COORD = 10000, 32, 128, 3
BN = 400


def _body(xT_ref, hh_ref, tT_ref, e_ref, W1_ref, b1_ref, W2_ref, b2_ref,
          coordT_ref, h_ref):
    @pl.when(pl.program_id(0) == 0)
    def _():
        t = jnp.clip(tT_ref[...], -1000.0, 1000.0)
        m = jnp.sum(t, axis=1) * (1.0 / DEG)
        coordT_ref[...] = jnp.clip(xT_ref[...], -1000.0, 1000.0) + m

    ef = jnp.sum(e_ref[...].reshape(BN, DEG, D), axis=1)
    hh = hh_ref[...]
    W1 = W1_ref[...]
    h1 = (jnp.dot(hh, W1[:D, :], preferred_element_type=jnp.float32)
          + jnp.dot(ef, W1[D:, :], preferred_element_type=jnp.float32)
          + b1_ref[...])
    h1 = h1 * jax.nn.sigmoid(h1)
    h_ref[...] = (hh
                  + jnp.dot(h1, W2_ref[...], preferred_element_type=jnp.float32)
                  + b2_ref[...])


def kernel(x, hh, trans, edge_feature, W1, b1, W2, b2):
    xT = x.T
    tT = trans.transpose(2, 1, 0)
    e2 = edge_feature.reshape(N * DEG, D)
    coordT, h = pl.pallas_call(
        _body,
        grid=(N // BN,),
        in_specs=[
            pl.BlockSpec((COORD, N), lambda i: (0, 0)),
            pl.BlockSpec((BN, D), lambda i: (i, 0)),
            pl.BlockSpec((COORD, DEG, N), lambda i: (0, 0, 0)),
            pl.BlockSpec((BN * DEG, D), lambda i: (i, 0)),
            pl.BlockSpec((2 * D, D), lambda i: (0, 0)),
            pl.BlockSpec((1, D), lambda i: (0, 0)),
            pl.BlockSpec((D, D), lambda i: (0, 0)),
            pl.BlockSpec((1, D), lambda i: (0, 0)),
        ],
        out_specs=[
            pl.BlockSpec((COORD, N), lambda i: (0, 0)),
            pl.BlockSpec((BN, D), lambda i: (i, 0)),
        ],
        out_shape=[
            jax.ShapeDtypeStruct((COORD, N), jnp.float32),
            jax.ShapeDtypeStruct((N, D), jnp.float32),
        ],
        compiler_params=pltpu.CompilerParams(
            dimension_semantics=("parallel",),
        ),
    )(xT, hh, tT, e2, W1, b1.reshape(1, D), W2, b2.reshape(1, D))
    return coordT.T, h
